# baseline (speedup 1.0000x reference)
import dataclasses
import functools

import jax
import jax.numpy as jnp
from jax import lax
from jax.experimental import pallas as pl
from jax.experimental.pallas import tpu as pltpu
from jax.experimental.pallas import tpu_sc as plsc

N = 10000
D = 128
H = 64
C = 2
E = 320000
EPS = 1e-5

NC = 2
NS = 16
NW = NC * NS
CHUNK = 128
EPT = 10240
E_PAD = NW * EPT
NCHUNK = EPT // CHUNK
ROWS_PER_SUB = 632
N_PAD = NS * ROWS_PER_SUB
CW = 16
NBUF = 5
LEAD = 3

BN_ROWS = 1000


def _seg_sum_sc(values, src_t, dst_t, z_acc, with_cnt):
  mesh = plsc.VectorSubcoreMesh(core_axis_name="c", subcore_axis_name="s")

  out_type = [jax.ShapeDtypeStruct((NC, N_PAD, H), jnp.float32)]
  scratch = [
      pltpu.VMEM((NCHUNK, CHUNK), jnp.int32),
      pltpu.VMEM((NCHUNK, CHUNK), jnp.int32),
      pltpu.VMEM((NBUF, CHUNK, H), jnp.float32),
      pltpu.SemaphoreType.DMA((NBUF,)),
      pltpu.SemaphoreType.DMA((NBUF,)),
      pltpu.VMEM_SHARED((N_PAD, H), jnp.float32),
  ]
  if with_cnt:
    out_type.append(jax.ShapeDtypeStruct((NW, N_PAD), jnp.float32))
    scratch.append(pltpu.VMEM((N_PAD,), jnp.float32))

  def body(vals_hbm, src_hbm, dst_hbm, zacc_hbm, *refs):
    if with_cnt:
      out_hbm, cnt_hbm, srcv, dstv, rows, gsem, ssem, acc, hist = refs
    else:
      out_hbm, srcv, dstv, rows, gsem, ssem, acc = refs
    cid = lax.axis_index("c")
    sid = lax.axis_index("s")
    wid = cid * NS + sid
    rstart = sid * ROWS_PER_SUB

    pltpu.sync_copy(src_hbm.at[wid], srcv)
    pltpu.sync_copy(dst_hbm.at[wid], dstv)

    pltpu.sync_copy(zacc_hbm.at[pl.ds(rstart, ROWS_PER_SUB)],
                    acc.at[pl.ds(rstart, ROWS_PER_SUB)])
    if with_cnt:
      zero16 = jnp.zeros((16,), jnp.float32)

      @pl.loop(0, N_PAD, step=16)
      def _(r):
        hist[pl.ds(r, 16)] = zero16

    plsc.subcore_barrier()

    def gather_start(c, b):
      pltpu.async_copy(vals_hbm.at[srcv.at[c]], rows.at[b], gsem.at[b])

    def gather_wait(c, b):
      pltpu.make_async_copy(vals_hbm.at[srcv.at[c]], rows.at[b],
                            gsem.at[b]).wait()

    def scatter_start(c, b):
      pltpu.async_copy(rows.at[b], acc.at[dstv.at[c]], ssem.at[b], add=True)

    def scatter_wait(c, b):
      pltpu.make_async_copy(rows.at[b], acc.at[dstv.at[c]], ssem.at[b]).wait()

    for b in range(LEAD):
      gather_start(b, b)

    @pl.loop(0, NCHUNK, step=NBUF)
    def _(c0):
      for b in range(NBUF):
        c = c0 + b
        nb = (b + LEAD) % NBUF
        nc = c + LEAD

        @pl.when(jnp.logical_and(nc >= NBUF, nc < NCHUNK))
        def _():
          scatter_wait(nc - NBUF, nb)

        @pl.when(jnp.logical_and(nc >= LEAD, nc < NCHUNK))
        def _():
          gather_start(nc, nb)

        gather_wait(c, b)
        scatter_start(c, b)
        if with_cnt:
          ones16 = jnp.full((16,), 1.0, jnp.float32)
          for j in range(CHUNK // 16):
            d16 = dstv[c, pl.ds(j * 16, 16)]
            plsc.addupdate_scatter(hist, [d16], ones16)

    for b in range(NBUF):
      scatter_wait(NCHUNK - NBUF + b, b)

    plsc.subcore_barrier()

    pltpu.sync_copy(acc.at[pl.ds(rstart, ROWS_PER_SUB)],
                    out_hbm.at[cid, pl.ds(rstart, ROWS_PER_SUB)])
    if with_cnt:
      pltpu.sync_copy(hist, cnt_hbm.at[wid])

  cp = pltpu.CompilerParams(use_tc_tiling_on_sc=False)
  if with_cnt and "needs_layout_passes" in pltpu.CompilerParams.__dataclass_fields__:
    cp = dataclasses.replace(cp, needs_layout_passes=False)
  k = pl.kernel(body, out_type=tuple(out_type), mesh=mesh,
                scratch_types=scratch, compiler_params=cp)
  return k(values, src_t, dst_t, z_acc)


def _dot(a, b):
  return jax.lax.dot(a, b, precision=lax.Precision.HIGHEST)


def _pre_tc(x, w_l, w_r):
  d_in = x.shape[1]

  def body(x_ref, wl_ref, wr_ref, a_ref, b_ref):
    xv = x_ref[...]
    a_ref[...] = _dot(xv, wl_ref[...])
    b_ref[...] = _dot(xv, wr_ref[...])

  return pl.pallas_call(
      body,
      grid=(N // BN_ROWS,),
      in_specs=[
          pl.BlockSpec((BN_ROWS, d_in), lambda i: (i, 0)),
          pl.BlockSpec((d_in, H), lambda i: (0, 0)),
          pl.BlockSpec((d_in, H), lambda i: (0, 0)),
      ],
      out_specs=[
          pl.BlockSpec((BN_ROWS, H), lambda i: (i, 0)),
          pl.BlockSpec((BN_ROWS, H), lambda i: (i, 0)),
      ],
      out_shape=[jax.ShapeDtypeStruct((N, H), jnp.float32)] * 2,
  )(x, w_l, w_r)


def _mid_tc(aggp, cntp, b_side, h_prev, bvec, svec, tvec, wl_n, wr_n, resid):
  def body(*refs):
    if resid:
      (a0, a1, cp, bs, hp, bv, sv, tv, wl, wr, h_ref, a_ref, b_ref) = refs
    else:
      (a0, a1, cp, bs, bv, sv, tv, wl, wr, h_ref, a_ref, b_ref) = refs
    cnt = jnp.maximum(jnp.sum(cp[...], axis=1, keepdims=True), 1.0)
    mean = (a0[0] + a1[0]) / cnt
    y = (mean + bs[...] + bv[...]) * sv[...] + tv[...]
    h = jnp.maximum(y, 0.0)
    if resid:
      h = h + hp[...]
    h_ref[...] = h
    a_ref[...] = _dot(h, wl[...])
    b_ref[...] = _dot(h, wr[...])

  blk3h = pl.BlockSpec((1, BN_ROWS, H), lambda i: (0, i, 0))
  blk3h1 = pl.BlockSpec((1, BN_ROWS, H), lambda i: (1, i, 0))
  blkc = pl.BlockSpec((BN_ROWS, NW), lambda i: (i, 0))
  blkh = pl.BlockSpec((BN_ROWS, H), lambda i: (i, 0))
  blkv = pl.BlockSpec((1, H), lambda i: (0, 0))
  blkw = pl.BlockSpec((H, H), lambda i: (0, 0))

  in_specs = [blk3h, blk3h1, blkc, blkh]
  args = [aggp, aggp, cntp, b_side]
  if resid:
    in_specs.append(blkh)
    args.append(h_prev)
  in_specs += [blkv, blkv, blkv, blkw, blkw]
  args += [bvec, svec, tvec, wl_n, wr_n]

  return pl.pallas_call(
      body,
      grid=(N // BN_ROWS,),
      in_specs=in_specs,
      out_specs=[blkh, blkh, blkh],
      out_shape=[jax.ShapeDtypeStruct((N, H), jnp.float32)] * 3,
  )(*args)


def _fin_tc(aggp, cntp, b_side, h_prev, bvec, svec, tvec, wc, bc):
  def body(a0, a1, cp, bs, hp, bv, sv, tv, wc_ref, bc_ref, o_ref):
    cnt = jnp.maximum(jnp.sum(cp[...], axis=1, keepdims=True), 1.0)
    mean = (a0[0] + a1[0]) / cnt
    y = (mean + bs[...] + bv[...]) * sv[...] + tv[...]
    h = jnp.maximum(y, 0.0) + hp[...]
    logits = _dot(h, wc_ref[...]) + bc_ref[...]
    m = jnp.max(logits, axis=1, keepdims=True)
    lse = m + jnp.log(jnp.sum(jnp.exp(logits - m), axis=1, keepdims=True))
    o_ref[...] = logits - lse

  blk3h = pl.BlockSpec((1, BN_ROWS, H), lambda i: (0, i, 0))
  blk3h1 = pl.BlockSpec((1, BN_ROWS, H), lambda i: (1, i, 0))
  blkc = pl.BlockSpec((BN_ROWS, NW), lambda i: (i, 0))
  blkh = pl.BlockSpec((BN_ROWS, H), lambda i: (i, 0))
  blkv = pl.BlockSpec((1, H), lambda i: (0, 0))

  return pl.pallas_call(
      body,
      grid=(N // BN_ROWS,),
      in_specs=[
          blk3h, blk3h1, blkc, blkh, blkh,
          blkv, blkv, blkv,
          pl.BlockSpec((H, C), lambda i: (0, 0)),
          pl.BlockSpec((1, C), lambda i: (0, 0)),
      ],
      out_specs=pl.BlockSpec((BN_ROWS, C), lambda i: (i, 0)),
      out_shape=jax.ShapeDtypeStruct((N, C), jnp.float32),
  )(aggp, aggp, cntp, b_side, h_prev, bvec, svec, tvec, wc, bc)


def kernel(x, edge_index, params):
  src = edge_index[0].astype(jnp.int32)
  dst = edge_index[1].astype(jnp.int32)
  pad = E_PAD - E
  src_t = jnp.concatenate([src, jnp.zeros((pad,), jnp.int32)]).reshape(
      NW, NCHUNK, CHUNK)
  dst_t = jnp.concatenate([dst, jnp.full((pad,), N, jnp.int32)]).reshape(
      NW, NCHUNK, CHUNK)
  z_acc = jnp.zeros((N_PAD, H), jnp.float32)

  k = 1.0 / jnp.sqrt(jnp.float32(1.0 + EPS))
  row = lambda v: v.reshape(1, -1)
  sv = [row(params[f'g{l}'] * k) for l in range(3)]
  tv = [row(params[f'bt{l}']) for l in range(3)]
  bv = [row(params[f'b{l}']) for l in range(3)]

  a0, b0 = _pre_tc(x, params['W0_l'], params['W0_r'])
  aggp, cntp = _seg_sum_sc(a0, src_t, dst_t, z_acc, with_cnt=True)
  cntp = cntp.T
  h1, a1, b1 = _mid_tc(aggp, cntp, b0, None, bv[0], sv[0], tv[0],
                       params['W1_l'], params['W1_r'], resid=False)
  (aggp1,) = _seg_sum_sc(a1, src_t, dst_t, z_acc, with_cnt=False)
  h2, a2, b2 = _mid_tc(aggp1, cntp, b1, h1, bv[1], sv[1], tv[1],
                       params['W2_l'], params['W2_r'], resid=True)
  (aggp2,) = _seg_sum_sc(a2, src_t, dst_t, z_acc, with_cnt=False)
  return _fin_tc(aggp2, cntp, b2, h2, bv[2], sv[2], tv[2],
                 params['Wc'], row(params['bc']))


# speedup vs baseline: 1.0011x; 1.0011x over previous
import dataclasses
import functools

import jax
import jax.numpy as jnp
from jax import lax
from jax.experimental import pallas as pl
from jax.experimental.pallas import tpu as pltpu
from jax.experimental.pallas import tpu_sc as plsc

N = 10000
D = 128
H = 64
C = 2
E = 320000
EPS = 1e-5

NC = 2
NS = 16
NW = NC * NS
CHUNK = 128
EPT = 10240
E_PAD = NW * EPT
NCHUNK = EPT // CHUNK
ROWS_PER_SUB = 632
N_PAD = NS * ROWS_PER_SUB
CW = 16
NBUF = 5
LEAD = 3

BN_ROWS = 1000


def _seg_sum_sc(values, src_t, dst_t, z_acc, with_cnt):
  mesh = plsc.VectorSubcoreMesh(core_axis_name="c", subcore_axis_name="s")

  SUP = 2 if with_cnt else 4
  NSUP = NCHUNK // SUP
  SUPC = SUP * CHUNK
  NB = 2
  LD = 1
  src_t = src_t.reshape(NW, NSUP, SUPC)
  dst_t = dst_t.reshape(NW, NSUP, SUPC)

  out_type = [jax.ShapeDtypeStruct((NC, N_PAD, H), jnp.float32)]
  scratch = [
      pltpu.VMEM((NSUP, SUPC), jnp.int32),
      pltpu.VMEM((NSUP, SUPC), jnp.int32),
      pltpu.VMEM((NB, SUPC, H), jnp.float32),
      pltpu.SemaphoreType.DMA((NB,)),
      pltpu.SemaphoreType.DMA((NB,)),
      pltpu.VMEM_SHARED((N_PAD, H), jnp.float32),
  ]
  if with_cnt:
    out_type.append(jax.ShapeDtypeStruct((NW, N_PAD), jnp.float32))
    scratch.append(pltpu.VMEM((N_PAD,), jnp.float32))

  def body(vals_hbm, src_hbm, dst_hbm, zacc_hbm, *refs):
    if with_cnt:
      out_hbm, cnt_hbm, srcv, dstv, rows, gsem, ssem, acc, hist = refs
    else:
      out_hbm, srcv, dstv, rows, gsem, ssem, acc = refs
    cid = lax.axis_index("c")
    sid = lax.axis_index("s")
    wid = cid * NS + sid
    rstart = sid * ROWS_PER_SUB

    pltpu.sync_copy(src_hbm.at[wid], srcv)
    pltpu.sync_copy(dst_hbm.at[wid], dstv)

    pltpu.sync_copy(zacc_hbm.at[pl.ds(rstart, ROWS_PER_SUB)],
                    acc.at[pl.ds(rstart, ROWS_PER_SUB)])
    if with_cnt:
      zero16 = jnp.zeros((16,), jnp.float32)

      @pl.loop(0, N_PAD, step=16)
      def _(r):
        hist[pl.ds(r, 16)] = zero16

    plsc.subcore_barrier()

    def sidx(refv, c):
      return refv.at[c]

    def gather_start(c, b):
      pltpu.async_copy(vals_hbm.at[sidx(srcv, c)], rows.at[b], gsem.at[b])

    def gather_wait(c, b):
      pltpu.make_async_copy(vals_hbm.at[sidx(srcv, c)], rows.at[b],
                            gsem.at[b]).wait()

    def scatter_start(c, b):
      pltpu.async_copy(rows.at[b], acc.at[sidx(dstv, c)], ssem.at[b],
                       add=True)

    def scatter_wait(c, b):
      pltpu.make_async_copy(rows.at[b], acc.at[sidx(dstv, c)],
                            ssem.at[b]).wait()

    for b in range(LD):
      gather_start(b, b)

    @pl.loop(0, NSUP, step=NB)
    def _(c0):
      for b in range(NB):
        c = c0 + b
        nb = (b + LD) % NB
        nc = c + LD

        @pl.when(jnp.logical_and(nc >= NB, nc < NSUP))
        def _():
          scatter_wait(nc - NB, nb)

        @pl.when(jnp.logical_and(nc >= LD, nc < NSUP))
        def _():
          gather_start(nc, nb)

        gather_wait(c, b)
        scatter_start(c, b)
        if with_cnt:
          ones16 = jnp.full((16,), 1.0, jnp.float32)
          for j in range(SUPC // 16):
            d16 = dstv[c, pl.ds(j * 16, 16)]
            plsc.addupdate_scatter(hist, [d16], ones16)

    for b in range(NB):
      scatter_wait(NSUP - NB + b, b)

    plsc.subcore_barrier()

    pltpu.sync_copy(acc.at[pl.ds(rstart, ROWS_PER_SUB)],
                    out_hbm.at[cid, pl.ds(rstart, ROWS_PER_SUB)])
    if with_cnt:
      pltpu.sync_copy(hist, cnt_hbm.at[wid])

  cp = pltpu.CompilerParams(use_tc_tiling_on_sc=False)
  if with_cnt and "needs_layout_passes" in pltpu.CompilerParams.__dataclass_fields__:
    cp = dataclasses.replace(cp, needs_layout_passes=False)
  k = pl.kernel(body, out_type=tuple(out_type), mesh=mesh,
                scratch_types=scratch, compiler_params=cp)
  return k(values, src_t, dst_t, z_acc)


def _dot(a, b):
  return jax.lax.dot(a, b, precision=lax.Precision.HIGHEST)


def _pre_tc(x, w_l, w_r):
  d_in = x.shape[1]

  def body(x_ref, wl_ref, wr_ref, a_ref, b_ref):
    xv = x_ref[...]
    a_ref[...] = _dot(xv, wl_ref[...])
    b_ref[...] = _dot(xv, wr_ref[...])

  return pl.pallas_call(
      body,
      grid=(N // BN_ROWS,),
      in_specs=[
          pl.BlockSpec((BN_ROWS, d_in), lambda i: (i, 0)),
          pl.BlockSpec((d_in, H), lambda i: (0, 0)),
          pl.BlockSpec((d_in, H), lambda i: (0, 0)),
      ],
      out_specs=[
          pl.BlockSpec((BN_ROWS, H), lambda i: (i, 0)),
          pl.BlockSpec((BN_ROWS, H), lambda i: (i, 0)),
      ],
      out_shape=[jax.ShapeDtypeStruct((N, H), jnp.float32)] * 2,
  )(x, w_l, w_r)


def _mid_tc(aggp, cntp, b_side, h_prev, bvec, svec, tvec, wl_n, wr_n, resid):
  def body(*refs):
    if resid:
      (a0, a1, cp, bs, hp, bv, sv, tv, wl, wr, h_ref, a_ref, b_ref) = refs
    else:
      (a0, a1, cp, bs, bv, sv, tv, wl, wr, h_ref, a_ref, b_ref) = refs
    cnt = jnp.maximum(jnp.sum(cp[...], axis=1, keepdims=True), 1.0)
    mean = (a0[0] + a1[0]) / cnt
    y = (mean + bs[...] + bv[...]) * sv[...] + tv[...]
    h = jnp.maximum(y, 0.0)
    if resid:
      h = h + hp[...]
    h_ref[...] = h
    a_ref[...] = _dot(h, wl[...])
    b_ref[...] = _dot(h, wr[...])

  blk3h = pl.BlockSpec((1, BN_ROWS, H), lambda i: (0, i, 0))
  blk3h1 = pl.BlockSpec((1, BN_ROWS, H), lambda i: (1, i, 0))
  blkc = pl.BlockSpec((BN_ROWS, NW), lambda i: (i, 0))
  blkh = pl.BlockSpec((BN_ROWS, H), lambda i: (i, 0))
  blkv = pl.BlockSpec((1, H), lambda i: (0, 0))
  blkw = pl.BlockSpec((H, H), lambda i: (0, 0))

  in_specs = [blk3h, blk3h1, blkc, blkh]
  args = [aggp, aggp, cntp, b_side]
  if resid:
    in_specs.append(blkh)
    args.append(h_prev)
  in_specs += [blkv, blkv, blkv, blkw, blkw]
  args += [bvec, svec, tvec, wl_n, wr_n]

  return pl.pallas_call(
      body,
      grid=(N // BN_ROWS,),
      in_specs=in_specs,
      out_specs=[blkh, blkh, blkh],
      out_shape=[jax.ShapeDtypeStruct((N, H), jnp.float32)] * 3,
  )(*args)


def _fin_tc(aggp, cntp, b_side, h_prev, bvec, svec, tvec, wc, bc):
  def body(a0, a1, cp, bs, hp, bv, sv, tv, wc_ref, bc_ref, o_ref):
    cnt = jnp.maximum(jnp.sum(cp[...], axis=1, keepdims=True), 1.0)
    mean = (a0[0] + a1[0]) / cnt
    y = (mean + bs[...] + bv[...]) * sv[...] + tv[...]
    h = jnp.maximum(y, 0.0) + hp[...]
    logits = _dot(h, wc_ref[...]) + bc_ref[...]
    m = jnp.max(logits, axis=1, keepdims=True)
    lse = m + jnp.log(jnp.sum(jnp.exp(logits - m), axis=1, keepdims=True))
    o_ref[...] = logits - lse

  blk3h = pl.BlockSpec((1, BN_ROWS, H), lambda i: (0, i, 0))
  blk3h1 = pl.BlockSpec((1, BN_ROWS, H), lambda i: (1, i, 0))
  blkc = pl.BlockSpec((BN_ROWS, NW), lambda i: (i, 0))
  blkh = pl.BlockSpec((BN_ROWS, H), lambda i: (i, 0))
  blkv = pl.BlockSpec((1, H), lambda i: (0, 0))

  return pl.pallas_call(
      body,
      grid=(N // BN_ROWS,),
      in_specs=[
          blk3h, blk3h1, blkc, blkh, blkh,
          blkv, blkv, blkv,
          pl.BlockSpec((H, C), lambda i: (0, 0)),
          pl.BlockSpec((1, C), lambda i: (0, 0)),
      ],
      out_specs=pl.BlockSpec((BN_ROWS, C), lambda i: (i, 0)),
      out_shape=jax.ShapeDtypeStruct((N, C), jnp.float32),
  )(aggp, aggp, cntp, b_side, h_prev, bvec, svec, tvec, wc, bc)


def kernel(x, edge_index, params):
  src = edge_index[0].astype(jnp.int32)
  dst = edge_index[1].astype(jnp.int32)
  pad = E_PAD - E
  src_t = jnp.concatenate([src, jnp.zeros((pad,), jnp.int32)]).reshape(
      NW, NCHUNK, CHUNK)
  dst_t = jnp.concatenate([dst, jnp.full((pad,), N, jnp.int32)]).reshape(
      NW, NCHUNK, CHUNK)
  z_acc = jnp.zeros((N_PAD, H), jnp.float32)

  k = 1.0 / jnp.sqrt(jnp.float32(1.0 + EPS))
  row = lambda v: v.reshape(1, -1)
  sv = [row(params[f'g{l}'] * k) for l in range(3)]
  tv = [row(params[f'bt{l}']) for l in range(3)]
  bv = [row(params[f'b{l}']) for l in range(3)]

  a0, b0 = _pre_tc(x, params['W0_l'], params['W0_r'])
  aggp, cntp = _seg_sum_sc(a0, src_t, dst_t, z_acc, with_cnt=True)
  cntp = cntp.T
  h1, a1, b1 = _mid_tc(aggp, cntp, b0, None, bv[0], sv[0], tv[0],
                       params['W1_l'], params['W1_r'], resid=False)
  (aggp1,) = _seg_sum_sc(a1, src_t, dst_t, z_acc, with_cnt=False)
  h2, a2, b2 = _mid_tc(aggp1, cntp, b1, h1, bv[1], sv[1], tv[1],
                       params['W2_l'], params['W2_r'], resid=True)
  (aggp2,) = _seg_sum_sc(a2, src_t, dst_t, z_acc, with_cnt=False)
  return _fin_tc(aggp2, cntp, b2, h2, bv[2], sv[2], tv[2],
                 params['Wc'], row(params['bc']))


# speedup vs baseline: 2.2157x; 2.2133x over previous
import dataclasses
import functools

import jax
import jax.numpy as jnp
from jax import lax
from jax.experimental import pallas as pl
from jax.experimental.pallas import tpu as pltpu
from jax.experimental.pallas import tpu_sc as plsc

N = 10000
D = 128
H = 64
C = 2
E = 320000
EPS = 1e-5

NC = 2
NS = 16
NW = NC * NS
CHUNK = 128
EPT = 10240
E_PAD = NW * EPT
NCHUNK = EPT // CHUNK
ROWS_PER_SUB = 632
N_PAD = NS * ROWS_PER_SUB
CW = 16
NBUF = 2

BN_ROWS = 1000


def _seg_sum_sc(values_pad, src_t, dst_t, z_acc):
  mesh = plsc.VectorSubcoreMesh(core_axis_name="c", subcore_axis_name="s")

  out_type = jax.ShapeDtypeStruct((NC, N_PAD, H), jnp.float32)
  scratch = [
      pltpu.VMEM((NCHUNK, CHUNK), jnp.int32),
      pltpu.VMEM((NCHUNK, CHUNK), jnp.int32),
      pltpu.VMEM((NBUF, CHUNK, H), jnp.float32),
      pltpu.SemaphoreType.DMA((NBUF,)),
      pltpu.VMEM_SHARED((N_PAD, H), jnp.float32),
      pltpu.VMEM_SHARED((N_PAD, H), jnp.float32),
  ]

  def body(vals_hbm, src_hbm, dst_hbm, zacc_hbm, out_hbm,
           srcv, dstv, rows, gsem, tab, acc):
    cid = lax.axis_index("c")
    sid = lax.axis_index("s")
    wid = cid * NS + sid
    rstart = sid * ROWS_PER_SUB
    rsl = pl.ds(rstart, ROWS_PER_SUB)

    pltpu.sync_copy(src_hbm.at[wid], srcv)
    pltpu.sync_copy(dst_hbm.at[wid], dstv)

    pltpu.sync_copy(vals_hbm.at[rsl], tab.at[rsl])
    pltpu.sync_copy(zacc_hbm.at[rsl], acc.at[rsl])
    plsc.subcore_barrier()

    def gather_start(c, b):
      pltpu.async_copy(tab.at[srcv.at[c]], rows.at[b], gsem.at[b])

    def gather_wait(c, b):
      pltpu.make_async_copy(tab.at[srcv.at[c]], rows.at[b],
                            gsem.at[b]).wait()

    for b in range(NBUF):
      gather_start(b, b)

    @pl.loop(0, NCHUNK, step=NBUF)
    def _(c0):
      for b in range(NBUF):
        c = c0 + b
        gather_wait(c, b)
        pltpu.sync_copy(rows.at[b], acc.at[dstv.at[c]], add=True)

        @pl.when(c + NBUF < NCHUNK)
        def _():
          gather_start(c + NBUF, b)

    plsc.subcore_barrier()

    pltpu.sync_copy(acc.at[rsl], out_hbm.at[cid, rsl])

  cp = pltpu.CompilerParams(use_tc_tiling_on_sc=False)
  k = pl.kernel(body, out_type=out_type, mesh=mesh,
                scratch_types=scratch, compiler_params=cp)
  return k(values_pad, src_t, dst_t, z_acc)


def _hist_sc(dst_t):
  mesh = plsc.VectorSubcoreMesh(core_axis_name="c", subcore_axis_name="s")

  scratch = [
      pltpu.VMEM((NCHUNK, CHUNK), jnp.int32),
      pltpu.VMEM((N_PAD,), jnp.float32),
  ]

  def body(dst_hbm, cnt_hbm, dstv, hist):
    cid = lax.axis_index("c")
    sid = lax.axis_index("s")
    wid = cid * NS + sid
    pltpu.sync_copy(dst_hbm.at[wid], dstv)

    zero16 = jnp.zeros((16,), jnp.float32)

    @pl.loop(0, N_PAD, step=16)
    def _(r):
      hist[pl.ds(r, 16)] = zero16

    ones16 = jnp.full((16,), 1.0, jnp.float32)

    @pl.loop(0, NCHUNK)
    def _(c):
      for j in range(CHUNK // 16):
        d16 = dstv[c, pl.ds(j * 16, 16)]
        plsc.addupdate_scatter(hist, [d16], ones16)

    pltpu.sync_copy(hist, cnt_hbm.at[wid])

  cp = pltpu.CompilerParams(use_tc_tiling_on_sc=False)
  if "needs_layout_passes" in pltpu.CompilerParams.__dataclass_fields__:
    cp = dataclasses.replace(cp, needs_layout_passes=False)
  k = pl.kernel(body, out_type=jax.ShapeDtypeStruct((NW, N_PAD), jnp.float32),
                mesh=mesh, scratch_types=scratch, compiler_params=cp)
  return k(dst_t)


def _dot(a, b):
  return jax.lax.dot(a, b, precision=lax.Precision.HIGHEST)


def _pre_tc(x, w_l, w_r):
  d_in = x.shape[1]

  def body(x_ref, wl_ref, wr_ref, a_ref, b_ref):
    xv = x_ref[...]
    a_ref[...] = _dot(xv, wl_ref[...])
    b_ref[...] = _dot(xv, wr_ref[...])

  return pl.pallas_call(
      body,
      grid=(N // BN_ROWS,),
      in_specs=[
          pl.BlockSpec((BN_ROWS, d_in), lambda i: (i, 0)),
          pl.BlockSpec((d_in, H), lambda i: (0, 0)),
          pl.BlockSpec((d_in, H), lambda i: (0, 0)),
      ],
      out_specs=[
          pl.BlockSpec((BN_ROWS, H), lambda i: (i, 0)),
          pl.BlockSpec((BN_ROWS, H), lambda i: (i, 0)),
      ],
      out_shape=[jax.ShapeDtypeStruct((N_PAD, H), jnp.float32),
                 jax.ShapeDtypeStruct((N, H), jnp.float32)],
  )(x, w_l, w_r)


def _mid_tc(aggp, cntp, b_side, h_prev, bvec, svec, tvec, wl_n, wr_n, resid):
  def body(*refs):
    if resid:
      (a0, a1, cp, bs, hp, bv, sv, tv, wl, wr, h_ref, a_ref, b_ref) = refs
    else:
      (a0, a1, cp, bs, bv, sv, tv, wl, wr, h_ref, a_ref, b_ref) = refs
    cnt = jnp.maximum(jnp.sum(cp[...], axis=1, keepdims=True), 1.0)
    mean = (a0[0] + a1[0]) / cnt
    y = (mean + bs[...] + bv[...]) * sv[...] + tv[...]
    h = jnp.maximum(y, 0.0)
    if resid:
      h = h + hp[...]
    h_ref[...] = h
    a_ref[...] = _dot(h, wl[...])
    b_ref[...] = _dot(h, wr[...])

  blk3h = pl.BlockSpec((1, BN_ROWS, H), lambda i: (0, i, 0))
  blk3h1 = pl.BlockSpec((1, BN_ROWS, H), lambda i: (1, i, 0))
  blkc = pl.BlockSpec((BN_ROWS, NW), lambda i: (i, 0))
  blkh = pl.BlockSpec((BN_ROWS, H), lambda i: (i, 0))
  blkv = pl.BlockSpec((1, H), lambda i: (0, 0))
  blkw = pl.BlockSpec((H, H), lambda i: (0, 0))

  in_specs = [blk3h, blk3h1, blkc, blkh]
  args = [aggp, aggp, cntp, b_side]
  if resid:
    in_specs.append(blkh)
    args.append(h_prev)
  in_specs += [blkv, blkv, blkv, blkw, blkw]
  args += [bvec, svec, tvec, wl_n, wr_n]

  return pl.pallas_call(
      body,
      grid=(N // BN_ROWS,),
      in_specs=in_specs,
      out_specs=[blkh, blkh, blkh],
      out_shape=[jax.ShapeDtypeStruct((N, H), jnp.float32),
                 jax.ShapeDtypeStruct((N_PAD, H), jnp.float32),
                 jax.ShapeDtypeStruct((N, H), jnp.float32)],
  )(*args)


def _fin_tc(aggp, cntp, b_side, h_prev, bvec, svec, tvec, wc, bc):
  def body(a0, a1, cp, bs, hp, bv, sv, tv, wc_ref, bc_ref, o_ref):
    cnt = jnp.maximum(jnp.sum(cp[...], axis=1, keepdims=True), 1.0)
    mean = (a0[0] + a1[0]) / cnt
    y = (mean + bs[...] + bv[...]) * sv[...] + tv[...]
    h = jnp.maximum(y, 0.0) + hp[...]
    logits = _dot(h, wc_ref[...]) + bc_ref[...]
    m = jnp.max(logits, axis=1, keepdims=True)
    lse = m + jnp.log(jnp.sum(jnp.exp(logits - m), axis=1, keepdims=True))
    o_ref[...] = logits - lse

  blk3h = pl.BlockSpec((1, BN_ROWS, H), lambda i: (0, i, 0))
  blk3h1 = pl.BlockSpec((1, BN_ROWS, H), lambda i: (1, i, 0))
  blkc = pl.BlockSpec((BN_ROWS, NW), lambda i: (i, 0))
  blkh = pl.BlockSpec((BN_ROWS, H), lambda i: (i, 0))
  blkv = pl.BlockSpec((1, H), lambda i: (0, 0))

  return pl.pallas_call(
      body,
      grid=(N // BN_ROWS,),
      in_specs=[
          blk3h, blk3h1, blkc, blkh, blkh,
          blkv, blkv, blkv,
          pl.BlockSpec((H, C), lambda i: (0, 0)),
          pl.BlockSpec((1, C), lambda i: (0, 0)),
      ],
      out_specs=pl.BlockSpec((BN_ROWS, C), lambda i: (i, 0)),
      out_shape=jax.ShapeDtypeStruct((N, C), jnp.float32),
  )(aggp, aggp, cntp, b_side, h_prev, bvec, svec, tvec, wc, bc)


def kernel(x, edge_index, params):
  src = edge_index[0].astype(jnp.int32)
  dst = edge_index[1].astype(jnp.int32)
  pad = E_PAD - E
  src_t = jnp.concatenate([src, jnp.zeros((pad,), jnp.int32)]).reshape(
      NW, NCHUNK, CHUNK)
  dst_t = jnp.concatenate([dst, jnp.full((pad,), N, jnp.int32)]).reshape(
      NW, NCHUNK, CHUNK)
  z_acc = jnp.zeros((N_PAD, H), jnp.float32)

  k = 1.0 / jnp.sqrt(jnp.float32(1.0 + EPS))
  row = lambda v: v.reshape(1, -1)
  sv = [row(params[f'g{l}'] * k) for l in range(3)]
  tv = [row(params[f'bt{l}']) for l in range(3)]
  bv = [row(params[f'b{l}']) for l in range(3)]

  cntp = _hist_sc(dst_t).T

  a0, b0 = _pre_tc(x, params['W0_l'], params['W0_r'])
  aggp = _seg_sum_sc(a0, src_t, dst_t, z_acc)
  h1, a1, b1 = _mid_tc(aggp, cntp, b0, None, bv[0], sv[0], tv[0],
                       params['W1_l'], params['W1_r'], resid=False)
  aggp1 = _seg_sum_sc(a1, src_t, dst_t, z_acc)
  h2, a2, b2 = _mid_tc(aggp1, cntp, b1, h1, bv[1], sv[1], tv[1],
                       params['W2_l'], params['W2_r'], resid=True)
  aggp2 = _seg_sum_sc(a2, src_t, dst_t, z_acc)
  return _fin_tc(aggp2, cntp, b2, h2, bv[2], sv[2], tv[2],
                 params['Wc'], row(params['bc']))
